# Initial kernel scaffold; baseline (speedup 1.0000x reference)
import functools

import jax
import jax.numpy as jnp
from jax import lax
from jax.experimental import pallas as pl
from jax.experimental.pallas import tpu as pltpu
from jax.experimental.pallas import tpu_sc as plsc

B = 4
N = 2048
DIM = 128
K = 16
BN = B * N
BNK = BN * K

R_TOPK = 256


def _topk_body(crow_ref, call_ref, fidx_ref):
    b = pl.program_id(0)
    i = pl.program_id(1)
    crow = crow_ref[0]
    call = call_ref[0]
    dots = lax.dot_general(crow, call, (((1,), (0,)), ((), ())),
                           preferred_element_type=jnp.float32)
    sqi = jnp.sum(crow * crow, axis=1, keepdims=True)
    sqj = jnp.sum(call * call, axis=0, keepdims=True)
    d2 = sqi + sqj - 2.0 * dots
    dist = jnp.sqrt(jnp.maximum(d2, 0.0))
    rows = lax.broadcasted_iota(jnp.float32, (R_TOPK, N), 0) + (i * R_TOPK)
    cols = lax.broadcasted_iota(jnp.float32, (R_TOPK, N), 1)
    dist = jnp.where(cols == rows, dist + 1e6, dist)

    work = dist
    picked = []
    for _ in range(K):
        m = jnp.min(work, axis=1, keepdims=True)
        am = jnp.min(jnp.where(work == m, cols, jnp.float32(N)),
                     axis=1, keepdims=True)
        picked.append(am)
        work = jnp.where(cols == am, jnp.float32(1e30), work)
    idx = jnp.concatenate(picked, axis=1).astype(jnp.int32) + b * N
    fidx_ref[0] = idx


def _knn_topk(coords, coords_t):
    return pl.pallas_call(
        _topk_body,
        grid=(B, N // R_TOPK),
        in_specs=[
            pl.BlockSpec((1, R_TOPK, 3), lambda b, i: (b, i, 0)),
            pl.BlockSpec((1, 3, N), lambda b, i: (b, 0, 0)),
        ],
        out_specs=pl.BlockSpec((1, R_TOPK, K), lambda b, i: (b, i, 0)),
        out_shape=jax.ShapeDtypeStruct((B, N, K), jnp.int32),
    )(coords, coords_t)


R_PQ = 1024


def _pq_body(x_ref, wp_ref, wq_ref, b1_ref, p_ref, q_ref):
    x = x_ref[...]
    p_ref[...] = lax.dot_general(x, wp_ref[...], (((1,), (0,)), ((), ())),
                                 preferred_element_type=jnp.float32) + b1_ref[...]
    q_ref[...] = lax.dot_general(x, wq_ref[...], (((1,), (0,)), ((), ())),
                                 preferred_element_type=jnp.float32)


def _pq(x, wp, wq, b1r):
    return pl.pallas_call(
        _pq_body,
        grid=(BN // R_PQ,),
        in_specs=[
            pl.BlockSpec((R_PQ, DIM + 3), lambda i: (i, 0)),
            pl.BlockSpec((DIM + 3, DIM), lambda i: (0, 0)),
            pl.BlockSpec((DIM + 3, DIM), lambda i: (0, 0)),
            pl.BlockSpec((1, DIM), lambda i: (0, 0)),
        ],
        out_specs=[
            pl.BlockSpec((R_PQ, DIM), lambda i: (i, 0)),
            pl.BlockSpec((R_PQ, DIM), lambda i: (i, 0)),
        ],
        out_shape=[
            jax.ShapeDtypeStruct((BN, DIM), jnp.float32),
            jax.ShapeDtypeStruct((BN, DIM), jnp.float32),
        ],
    )(x, wp, wq, b1r)


SC_NC = 2
SC_NS = 16
SC_NW = SC_NC * SC_NS
IDX_ROWS = BNK // 128
ROWS_PER_W = IDX_ROWS // SC_NW


def _sc_gather_body(idx_hbm, q_hbm, out_hbm, idx_v, rows_v, sem):
    wid = lax.axis_index("s") * SC_NC + lax.axis_index("c")
    base = wid * ROWS_PER_W
    pltpu.sync_copy(idx_hbm.at[pl.ds(base, ROWS_PER_W)], idx_v)

    def body(r, carry):
        pltpu.async_copy(q_hbm.at[idx_v.at[r]], rows_v, sem).wait()
        pltpu.sync_copy(rows_v, out_hbm.at[pl.ds((base + r) * 128, 128)])
        return carry

    lax.fori_loop(0, ROWS_PER_W, body, 0)


_sc_gather = functools.partial(
    pl.kernel,
    mesh=plsc.VectorSubcoreMesh(core_axis_name="c", subcore_axis_name="s"),
    out_type=jax.ShapeDtypeStruct((BNK, DIM), jnp.float32),
    scratch_types=[
        pltpu.VMEM((ROWS_PER_W, 128), jnp.int32),
        pltpu.VMEM((128, DIM), jnp.float32),
        pltpu.SemaphoreType.DMA,
    ],
)(_sc_gather_body)


R_MLP = 128


def _mlp_body(qg_ref, p_ref, w2_ref, b2_ref, out_ref):
    x = qg_ref[...]
    pblk = p_ref[...]
    h = x.reshape(R_MLP, K, DIM) + pblk[:, None, :]
    h = jax.nn.gelu(h, approximate=False)
    h2 = lax.dot_general(h.reshape(R_MLP * K, DIM), w2_ref[...],
                         (((1,), (0,)), ((), ())),
                         preferred_element_type=jnp.float32) + b2_ref[...]
    h2 = jax.nn.gelu(h2, approximate=False)
    out_ref[...] = jnp.max(h2.reshape(R_MLP, K, DIM), axis=1)


def _mlp(qg, p, w2, b2r):
    return pl.pallas_call(
        _mlp_body,
        grid=(BN // R_MLP,),
        in_specs=[
            pl.BlockSpec((R_MLP * K, DIM), lambda i: (i, 0)),
            pl.BlockSpec((R_MLP, DIM), lambda i: (i, 0)),
            pl.BlockSpec((DIM, DIM), lambda i: (0, 0)),
            pl.BlockSpec((1, DIM), lambda i: (0, 0)),
        ],
        out_specs=pl.BlockSpec((R_MLP, DIM), lambda i: (i, 0)),
        out_shape=jax.ShapeDtypeStruct((BN, DIM), jnp.float32),
    )(qg, p, w2, b2r)


def kernel(feats, coords, W1, b1, W2, b2):
    coords_t = jnp.swapaxes(coords, 1, 2)
    fidx = _knn_topk(coords, coords_t)

    wa = W1[:DIM]
    wb = W1[DIM:2 * DIM]
    wc = W1[2 * DIM:]
    wp = jnp.concatenate([wa - wb, -wc], axis=0)
    wq = jnp.concatenate([wb, wc], axis=0)
    x = jnp.concatenate(
        [feats.reshape(BN, DIM), coords.reshape(BN, 3)], axis=1)
    p, q = _pq(x, wp, wq, b1.reshape(1, DIM))

    idx2 = fidx.reshape(IDX_ROWS, 128)
    qg = _sc_gather(idx2, q)

    out = _mlp(qg, p, W2, b2.reshape(1, DIM))
    return out.reshape(B, N, DIM)


# speedup vs baseline: 13.5762x; 13.5762x over previous
import functools

import jax
import jax.numpy as jnp
from jax import lax
from jax.experimental import pallas as pl
from jax.experimental.pallas import tpu as pltpu
from jax.experimental.pallas import tpu_sc as plsc

B = 4
N = 2048
DIM = 128
K = 16
BN = B * N
BNK = BN * K

R_TOPK = 256


def _topk_body(crow_ref, call_ref, fidx_ref):
    b = pl.program_id(0)
    i = pl.program_id(1)
    crow = crow_ref[0]
    call = call_ref[0]
    dots = lax.dot_general(crow, call, (((1,), (0,)), ((), ())),
                           preferred_element_type=jnp.float32)
    sqi = jnp.sum(crow * crow, axis=1, keepdims=True)
    sqj = jnp.sum(call * call, axis=0, keepdims=True)
    d2 = sqi + sqj - 2.0 * dots
    dist = jnp.sqrt(jnp.maximum(d2, 0.0))
    rows = lax.broadcasted_iota(jnp.int32, (R_TOPK, N), 0) + (i * R_TOPK)
    cols = lax.broadcasted_iota(jnp.int32, (R_TOPK, N), 1)
    dist = jnp.where(cols == rows, dist + 1e6, dist)

    work = dist
    picked = []
    for _ in range(K):
        m = jnp.min(work, axis=1, keepdims=True)
        am = jnp.min(jnp.where(work == m, cols, jnp.int32(N)),
                     axis=1, keepdims=True)
        picked.append(am)
        work = jnp.where(cols == am, jnp.float32(1e30), work)
    idx = jnp.concatenate(picked, axis=1) + b * N
    fidx_ref[0] = idx


def _knn_topk(coords, coords_t):
    return pl.pallas_call(
        _topk_body,
        grid=(B, N // R_TOPK),
        in_specs=[
            pl.BlockSpec((1, R_TOPK, 3), lambda b, i: (b, i, 0)),
            pl.BlockSpec((1, 3, N), lambda b, i: (b, 0, 0)),
        ],
        out_specs=pl.BlockSpec((1, R_TOPK, K), lambda b, i: (b, i, 0)),
        out_shape=jax.ShapeDtypeStruct((B, N, K), jnp.int32),
    )(coords, coords_t)


R_PQ = 1024


def _pq_body(x_ref, wp_ref, wq_ref, b1_ref, p_ref, q_ref):
    x = x_ref[...]
    p_ref[...] = lax.dot_general(x, wp_ref[...], (((1,), (0,)), ((), ())),
                                 preferred_element_type=jnp.float32) + b1_ref[...]
    q_ref[...] = lax.dot_general(x, wq_ref[...], (((1,), (0,)), ((), ())),
                                 preferred_element_type=jnp.float32)


def _pq(x, wp, wq, b1r):
    return pl.pallas_call(
        _pq_body,
        grid=(BN // R_PQ,),
        in_specs=[
            pl.BlockSpec((R_PQ, DIM + 3), lambda i: (i, 0)),
            pl.BlockSpec((DIM + 3, DIM), lambda i: (0, 0)),
            pl.BlockSpec((DIM + 3, DIM), lambda i: (0, 0)),
            pl.BlockSpec((1, DIM), lambda i: (0, 0)),
        ],
        out_specs=[
            pl.BlockSpec((R_PQ, DIM), lambda i: (i, 0)),
            pl.BlockSpec((R_PQ, DIM), lambda i: (i, 0)),
        ],
        out_shape=[
            jax.ShapeDtypeStruct((BN, DIM), jnp.float32),
            jax.ShapeDtypeStruct((BN, DIM), jnp.float32),
        ],
    )(x, wp, wq, b1r)


SC_NC = 2
SC_NS = 16
SC_NW = SC_NC * SC_NS
IDX_ROWS = BNK // 128
ROWS_PER_W = IDX_ROWS // SC_NW


def _sc_gather_body(idx_hbm, q_hbm, out_hbm, idx_v, rows_v, sem):
    wid = lax.axis_index("s") * SC_NC + lax.axis_index("c")
    base = wid * ROWS_PER_W
    pltpu.sync_copy(idx_hbm.at[pl.ds(base, ROWS_PER_W)], idx_v)

    def body(r, carry):
        pltpu.async_copy(q_hbm.at[idx_v.at[r]], rows_v, sem).wait()
        pltpu.sync_copy(rows_v, out_hbm.at[pl.ds((base + r) * 128, 128)])
        return carry

    lax.fori_loop(0, ROWS_PER_W, body, 0)


@functools.cache
def _sc_gather_kernel():
    return pl.kernel(
        _sc_gather_body,
        mesh=plsc.VectorSubcoreMesh(core_axis_name="c", subcore_axis_name="s"),
        out_type=jax.ShapeDtypeStruct((BNK, DIM), jnp.float32),
        scratch_types=[
            pltpu.VMEM((ROWS_PER_W, 128), jnp.int32),
            pltpu.VMEM((128, DIM), jnp.float32),
            pltpu.SemaphoreType.DMA,
        ],
    )


def _sc_gather(idx2, q):
    return _sc_gather_kernel()(idx2, q)


R_MLP = 128


def _gelu_exact(x):
    return 0.5 * x * (1.0 + lax.erf(x * jnp.float32(0.7071067811865476)))


def _mlp_body(qg_ref, p_ref, w2_ref, b2_ref, out_ref):
    x = qg_ref[...]
    pblk = p_ref[...]
    h = x.reshape(R_MLP, K, DIM) + pblk[:, None, :]
    h = _gelu_exact(h)
    h2 = lax.dot_general(h.reshape(R_MLP * K, DIM), w2_ref[...],
                         (((1,), (0,)), ((), ())),
                         preferred_element_type=jnp.float32) + b2_ref[...]
    h2 = _gelu_exact(h2)
    out_ref[...] = jnp.max(h2.reshape(R_MLP, K, DIM), axis=1)


def _mlp(qg, p, w2, b2r):
    return pl.pallas_call(
        _mlp_body,
        grid=(BN // R_MLP,),
        in_specs=[
            pl.BlockSpec((R_MLP * K, DIM), lambda i: (i, 0)),
            pl.BlockSpec((R_MLP, DIM), lambda i: (i, 0)),
            pl.BlockSpec((DIM, DIM), lambda i: (0, 0)),
            pl.BlockSpec((1, DIM), lambda i: (0, 0)),
        ],
        out_specs=pl.BlockSpec((R_MLP, DIM), lambda i: (i, 0)),
        out_shape=jax.ShapeDtypeStruct((BN, DIM), jnp.float32),
    )(qg, p, w2, b2r)


def kernel(feats, coords, W1, b1, W2, b2):
    coords_t = jnp.swapaxes(coords, 1, 2)
    fidx = _knn_topk(coords, coords_t)

    wa = W1[:DIM]
    wb = W1[DIM:2 * DIM]
    wc = W1[2 * DIM:]
    wp = jnp.concatenate([wa - wb, -wc], axis=0)
    wq = jnp.concatenate([wb, wc], axis=0)
    x = jnp.concatenate(
        [feats.reshape(BN, DIM), coords.reshape(BN, 3)], axis=1)
    p, q = _pq(x, wp, wq, b1.reshape(1, DIM))

    idx2 = fidx.reshape(IDX_ROWS, 128)
    qg = _sc_gather(idx2, q)

    out = _mlp(qg, p, W2, b2.reshape(1, DIM))
    return out.reshape(B, N, DIM)


# speedup vs baseline: 15.1903x; 1.1189x over previous
import functools

import jax
import jax.numpy as jnp
from jax import lax
from jax.experimental import pallas as pl
from jax.experimental.pallas import tpu as pltpu
from jax.experimental.pallas import tpu_sc as plsc

B = 4
N = 2048
DIM = 128
K = 16
BN = B * N
BNK = BN * K

R_TOPK = 256


def _topk_body(crow_ref, call_ref, fidx_ref):
    b = pl.program_id(0)
    i = pl.program_id(1)
    crow = crow_ref[0]
    call = call_ref[0]
    dots = lax.dot_general(crow, call, (((1,), (0,)), ((), ())),
                           preferred_element_type=jnp.float32)
    sqi = jnp.sum(crow * crow, axis=1, keepdims=True)
    sqj = jnp.sum(call * call, axis=0, keepdims=True)
    d2 = sqi + sqj - 2.0 * dots
    dist = jnp.sqrt(jnp.maximum(d2, 0.0))
    rows = lax.broadcasted_iota(jnp.int32, (R_TOPK, N), 0) + (i * R_TOPK)
    cols = lax.broadcasted_iota(jnp.int32, (R_TOPK, N), 1)
    dist = jnp.where(cols == rows, dist + 1e6, dist)

    work = dist
    picked = []
    for _ in range(K):
        am = jnp.argmin(work, axis=1).astype(jnp.int32)[:, None]
        picked.append(am)
        work = jnp.where(cols == am, jnp.float32(1e30), work)
    idx = jnp.concatenate(picked, axis=1) + b * N
    fidx_ref[0] = idx


def _knn_topk(coords, coords_t):
    return pl.pallas_call(
        _topk_body,
        grid=(B, N // R_TOPK),
        in_specs=[
            pl.BlockSpec((1, R_TOPK, 3), lambda b, i: (b, i, 0)),
            pl.BlockSpec((1, 3, N), lambda b, i: (b, 0, 0)),
        ],
        out_specs=pl.BlockSpec((1, R_TOPK, K), lambda b, i: (b, i, 0)),
        out_shape=jax.ShapeDtypeStruct((B, N, K), jnp.int32),
    )(coords, coords_t)


R_PQ = 1024


def _pq_body(x_ref, wp_ref, wq_ref, b1_ref, p_ref, q_ref):
    x = x_ref[...]
    p_ref[...] = lax.dot_general(x, wp_ref[...], (((1,), (0,)), ((), ())),
                                 preferred_element_type=jnp.float32) + b1_ref[...]
    q_ref[...] = lax.dot_general(x, wq_ref[...], (((1,), (0,)), ((), ())),
                                 preferred_element_type=jnp.float32)


def _pq(x, wp, wq, b1r):
    return pl.pallas_call(
        _pq_body,
        grid=(BN // R_PQ,),
        in_specs=[
            pl.BlockSpec((R_PQ, DIM + 3), lambda i: (i, 0)),
            pl.BlockSpec((DIM + 3, DIM), lambda i: (0, 0)),
            pl.BlockSpec((DIM + 3, DIM), lambda i: (0, 0)),
            pl.BlockSpec((1, DIM), lambda i: (0, 0)),
        ],
        out_specs=[
            pl.BlockSpec((R_PQ, DIM), lambda i: (i, 0)),
            pl.BlockSpec((R_PQ, DIM), lambda i: (i, 0)),
        ],
        out_shape=[
            jax.ShapeDtypeStruct((BN, DIM), jnp.float32),
            jax.ShapeDtypeStruct((BN, DIM), jnp.float32),
        ],
    )(x, wp, wq, b1r)


SC_NC = 2
SC_NS = 16
SC_NW = SC_NC * SC_NS
IDX_ROWS = BNK // 128
ROWS_PER_W = IDX_ROWS // SC_NW


def _sc_gather_body(idx_hbm, q_hbm, out_hbm, idx_v, rows_v, sem):
    wid = lax.axis_index("s") * SC_NC + lax.axis_index("c")
    base = wid * ROWS_PER_W
    pltpu.sync_copy(idx_hbm.at[pl.ds(base, ROWS_PER_W)], idx_v)

    def body(r, carry):
        pltpu.async_copy(q_hbm.at[idx_v.at[r]], rows_v, sem).wait()
        pltpu.sync_copy(rows_v, out_hbm.at[pl.ds((base + r) * 128, 128)])
        return carry

    lax.fori_loop(0, ROWS_PER_W, body, 0)


@functools.cache
def _sc_gather_kernel():
    return pl.kernel(
        _sc_gather_body,
        mesh=plsc.VectorSubcoreMesh(core_axis_name="c", subcore_axis_name="s"),
        out_type=jax.ShapeDtypeStruct((BNK, DIM), jnp.float32),
        scratch_types=[
            pltpu.VMEM((ROWS_PER_W, 128), jnp.int32),
            pltpu.VMEM((128, DIM), jnp.float32),
            pltpu.SemaphoreType.DMA,
        ],
    )


def _sc_gather(idx2, q):
    return _sc_gather_kernel()(idx2, q)


R_MLP = 128


def _gelu_exact(x):
    return 0.5 * x * (1.0 + lax.erf(x * jnp.float32(0.7071067811865476)))


def _mlp_body(qg_ref, p_ref, w2_ref, b2_ref, out_ref):
    x = qg_ref[...]
    pblk = p_ref[...]
    h = x.reshape(R_MLP, K, DIM) + pblk[:, None, :]
    h = _gelu_exact(h)
    h2 = lax.dot_general(h.reshape(R_MLP * K, DIM), w2_ref[...],
                         (((1,), (0,)), ((), ())),
                         preferred_element_type=jnp.float32) + b2_ref[...]
    z = h2.reshape(R_MLP, K, DIM)
    zmax = jnp.max(z, axis=1)
    zmin = jnp.min(z, axis=1)
    out_ref[...] = jnp.maximum(_gelu_exact(zmax), _gelu_exact(zmin))


def _mlp(qg, p, w2, b2r):
    return pl.pallas_call(
        _mlp_body,
        grid=(BN // R_MLP,),
        in_specs=[
            pl.BlockSpec((R_MLP * K, DIM), lambda i: (i, 0)),
            pl.BlockSpec((R_MLP, DIM), lambda i: (i, 0)),
            pl.BlockSpec((DIM, DIM), lambda i: (0, 0)),
            pl.BlockSpec((1, DIM), lambda i: (0, 0)),
        ],
        out_specs=pl.BlockSpec((R_MLP, DIM), lambda i: (i, 0)),
        out_shape=jax.ShapeDtypeStruct((BN, DIM), jnp.float32),
    )(qg, p, w2, b2r)


def kernel(feats, coords, W1, b1, W2, b2):
    coords_t = jnp.swapaxes(coords, 1, 2)
    fidx = _knn_topk(coords, coords_t)

    wa = W1[:DIM]
    wb = W1[DIM:2 * DIM]
    wc = W1[2 * DIM:]
    wp = jnp.concatenate([wa - wb, -wc], axis=0)
    wq = jnp.concatenate([wb, wc], axis=0)
    x = jnp.concatenate(
        [feats.reshape(BN, DIM), coords.reshape(BN, 3)], axis=1)
    p, q = _pq(x, wp, wq, b1.reshape(1, DIM))

    idx2 = fidx.reshape(IDX_ROWS, 128)
    qg = _sc_gather(idx2, q)

    out = _mlp(qg, p, W2, b2.reshape(1, DIM))
    return out.reshape(B, N, DIM)


# speedup vs baseline: 16.8450x; 1.1089x over previous
import functools

import jax
import jax.numpy as jnp
from jax import lax
from jax.experimental import pallas as pl
from jax.experimental.pallas import tpu as pltpu
from jax.experimental.pallas import tpu_sc as plsc

B = 4
N = 2048
DIM = 128
K = 16
BN = B * N
BNK = BN * K

R_TOPK = 256


def _topk_body(boff, crow_ref, call_ref, fidx_ref):
    i = pl.program_id(0)
    crow = crow_ref[...]
    call = call_ref[...]
    dots = lax.dot_general(crow, call, (((1,), (0,)), ((), ())),
                           preferred_element_type=jnp.float32)
    sqi = jnp.sum(crow * crow, axis=1, keepdims=True)
    sqj = jnp.sum(call * call, axis=0, keepdims=True)
    d2 = sqi + sqj - 2.0 * dots
    dist = jnp.sqrt(jnp.maximum(d2, 0.0))
    rows = lax.broadcasted_iota(jnp.int32, (R_TOPK, N), 0) + (i * R_TOPK)
    cols = lax.broadcasted_iota(jnp.int32, (R_TOPK, N), 1)
    dist = jnp.where(cols == rows, dist + 1e6, dist)

    work = dist
    picked = []
    for _ in range(K):
        am = jnp.argmin(work, axis=1).astype(jnp.int32)[:, None]
        picked.append(am)
        work = jnp.where(cols == am, jnp.float32(1e30), work)
    idx = jnp.concatenate(picked, axis=1) + boff
    fidx_ref[...] = idx


def _knn_topk_b(coords_b, coords_t_b, boff):
    return pl.pallas_call(
        functools.partial(_topk_body, boff),
        grid=(N // R_TOPK,),
        in_specs=[
            pl.BlockSpec((R_TOPK, 3), lambda i: (i, 0)),
            pl.BlockSpec((3, N), lambda i: (0, 0)),
        ],
        out_specs=pl.BlockSpec((R_TOPK, K), lambda i: (i, 0)),
        out_shape=jax.ShapeDtypeStruct((N, K), jnp.int32),
    )(coords_b, coords_t_b)


R_PQ = 1024


def _pq_body(x_ref, wp_ref, wq_ref, b1_ref, p_ref, q_ref):
    x = x_ref[...]
    p_ref[...] = lax.dot_general(x, wp_ref[...], (((1,), (0,)), ((), ())),
                                 preferred_element_type=jnp.float32) + b1_ref[...]
    q_ref[...] = lax.dot_general(x, wq_ref[...], (((1,), (0,)), ((), ())),
                                 preferred_element_type=jnp.float32)


def _pq(x, wp, wq, b1r):
    return pl.pallas_call(
        _pq_body,
        grid=(BN // R_PQ,),
        in_specs=[
            pl.BlockSpec((R_PQ, DIM + 3), lambda i: (i, 0)),
            pl.BlockSpec((DIM + 3, DIM), lambda i: (0, 0)),
            pl.BlockSpec((DIM + 3, DIM), lambda i: (0, 0)),
            pl.BlockSpec((1, DIM), lambda i: (0, 0)),
        ],
        out_specs=[
            pl.BlockSpec((R_PQ, DIM), lambda i: (i, 0)),
            pl.BlockSpec((R_PQ, DIM), lambda i: (i, 0)),
        ],
        out_shape=[
            jax.ShapeDtypeStruct((BN, DIM), jnp.float32),
            jax.ShapeDtypeStruct((BN, DIM), jnp.float32),
        ],
    )(x, wp, wq, b1r)


SC_NC = 2
SC_NS = 16
SC_NW = SC_NC * SC_NS


def _sc_gather_body(rows_per_w, idx_hbm, q_hbm, out_hbm, idx_v, rows_v, sem):
    wid = lax.axis_index("s") * SC_NC + lax.axis_index("c")
    base = wid * rows_per_w
    pltpu.sync_copy(idx_hbm.at[pl.ds(base, rows_per_w)], idx_v)

    def body(r, carry):
        pltpu.async_copy(q_hbm.at[idx_v.at[r]], rows_v, sem).wait()
        pltpu.sync_copy(rows_v, out_hbm.at[pl.ds((base + r) * 128, 128)])
        return carry

    lax.fori_loop(0, rows_per_w, body, 0)


@functools.cache
def _sc_gather_kernel(nidx):
    rows_per_w = nidx // 128 // SC_NW
    return pl.kernel(
        functools.partial(_sc_gather_body, rows_per_w),
        mesh=plsc.VectorSubcoreMesh(core_axis_name="c", subcore_axis_name="s"),
        out_type=jax.ShapeDtypeStruct((nidx, DIM), jnp.float32),
        scratch_types=[
            pltpu.VMEM((rows_per_w, 128), jnp.int32),
            pltpu.VMEM((128, DIM), jnp.float32),
            pltpu.SemaphoreType.DMA,
        ],
    )


def _sc_gather(idx2, q):
    return _sc_gather_kernel(idx2.shape[0] * 128)(idx2, q)


R_MLP = 128


def _gelu_exact(x):
    return 0.5 * x * (1.0 + lax.erf(x * jnp.float32(0.7071067811865476)))


def _mlp_body(qg_ref, p_ref, w2_ref, b2_ref, out_ref):
    x = qg_ref[...]
    pblk = p_ref[...]
    h = x.reshape(R_MLP, K, DIM) + pblk[:, None, :]
    h = _gelu_exact(h)
    h2 = lax.dot_general(h.reshape(R_MLP * K, DIM), w2_ref[...],
                         (((1,), (0,)), ((), ())),
                         preferred_element_type=jnp.float32) + b2_ref[...]
    z = h2.reshape(R_MLP, K, DIM)
    zmax = jnp.max(z, axis=1)
    zmin = jnp.min(z, axis=1)
    out_ref[...] = jnp.maximum(_gelu_exact(zmax), _gelu_exact(zmin))


def _mlp(qg, p, w2, b2r):
    return pl.pallas_call(
        _mlp_body,
        grid=(p.shape[0] // R_MLP,),
        in_specs=[
            pl.BlockSpec((R_MLP * K, DIM), lambda i: (i, 0)),
            pl.BlockSpec((R_MLP, DIM), lambda i: (i, 0)),
            pl.BlockSpec((DIM, DIM), lambda i: (0, 0)),
            pl.BlockSpec((1, DIM), lambda i: (0, 0)),
        ],
        out_specs=pl.BlockSpec((R_MLP, DIM), lambda i: (i, 0)),
        out_shape=jax.ShapeDtypeStruct((p.shape[0], DIM), jnp.float32),
    )(qg, p, w2, b2r)


def kernel(feats, coords, W1, b1, W2, b2):
    coords_t = jnp.swapaxes(coords, 1, 2)

    wa = W1[:DIM]
    wb = W1[DIM:2 * DIM]
    wc = W1[2 * DIM:]
    wp = jnp.concatenate([wa - wb, -wc], axis=0)
    wq = jnp.concatenate([wb, wc], axis=0)
    x = jnp.concatenate(
        [feats.reshape(BN, DIM), coords.reshape(BN, 3)], axis=1)
    p, q = _pq(x, wp, wq, b1.reshape(1, DIM))

    b2r = b2.reshape(1, DIM)
    outs = []
    qgs = []
    for b in range(B):
        fidx_b = _knn_topk_b(coords[b], coords_t[b], b * N)
        qgs.append(_sc_gather(fidx_b.reshape(N * K // 128, 128), q))
    for b in range(B):
        outs.append(_mlp(qgs[b], p[b * N:(b + 1) * N], W2, b2r))
    return jnp.stack(outs).reshape(B, N, DIM)
